# baseline (speedup 1.0000x reference)
import functools

import jax
import jax.numpy as jnp
from jax import lax
from jax.experimental import pallas as pl
from jax.experimental.pallas import tpu as pltpu
from jax.experimental.pallas import tpu_sc as plsc

VOCAB = 100000
DIM = 64
EMBED_P = 0.1
BATCH = 4096
HIST = 200

_B = BATCH * HIST

_info = plsc.get_sparse_core_info()
_NC = _info.num_cores
_NS = _info.num_subcores
_NW = _NC * _NS
_BPW = _B // _NW
_CH = 128
_NCHUNK = _BPW // _CH
_NBUF = 8
_LOOK = 4

_MROWS = 4000


def _scale_body(w_ref, m_ref, o_ref):
    m_row = m_ref[...].reshape(1, _MROWS)
    m_col = lax.transpose(m_row, (1, 0))
    o_ref[...] = w_ref[...] * m_col


def _masked_table(weight, mask_lanes):
    grid = VOCAB // _MROWS
    return pl.pallas_call(
        _scale_body,
        grid=(grid,),
        in_specs=[
            pl.BlockSpec((_MROWS, DIM), lambda i: (i, 0)),
            pl.BlockSpec((1, 1, _MROWS), lambda i: (i, 0, 0)),
        ],
        out_specs=pl.BlockSpec((_MROWS, DIM), lambda i: (i, 0)),
        out_shape=jax.ShapeDtypeStruct((VOCAB, DIM), jnp.float32),
    )(weight, mask_lanes)


_mesh = plsc.VectorSubcoreMesh(core_axis_name="c", subcore_axis_name="s")


@functools.partial(
    pl.kernel,
    mesh=_mesh,
    out_type=jax.ShapeDtypeStruct((_B, DIM), jnp.float32),
    scratch_types=[
        pltpu.VMEM((_NCHUNK, _CH), jnp.int32),
    ]
    + [pltpu.VMEM((_CH, DIM), jnp.float32) for _ in range(_NBUF)]
    + [pltpu.SemaphoreType.DMA for _ in range(2 * _NBUF)],
    compiler_params=pltpu.CompilerParams(use_tc_tiling_on_sc=False),
)
def _sc_gather(tab_hbm, idx_hbm, out_hbm, idx_v, *bufs_and_sems):
    rows = bufs_and_sems[:_NBUF]
    gsem = bufs_and_sems[_NBUF:2 * _NBUF]
    ssem = bufs_and_sems[2 * _NBUF:]
    wid = lax.axis_index("s") * _NC + lax.axis_index("c")
    base_c = wid * _NCHUNK
    pltpu.sync_copy(idx_hbm.at[wid], idx_v)

    def start_gather(j, b):
        pltpu.async_copy(tab_hbm.at[idx_v.at[j]], rows[b], gsem[b])

    def wait_gather(j, b):
        pltpu.make_async_copy(tab_hbm.at[idx_v.at[j]], rows[b], gsem[b]).wait()

    def _dst(j):
        c = base_c + j
        return (c // (BATCH // _CH)) * BATCH + (c % (BATCH // _CH)) * _CH

    def start_scatter(j, b):
        pltpu.async_copy(rows[b], out_hbm.at[pl.ds(_dst(j), _CH)], ssem[b])

    def wait_scatter(j, b):
        pltpu.make_async_copy(
            rows[b], out_hbm.at[pl.ds(_dst(j), _CH)], ssem[b]
        ).wait()

    for b in range(_LOOK):
        start_gather(b, b)

    for b in range(_NBUF):
        wait_gather(b, b)
        start_scatter(b, b)
        if b >= _LOOK:
            wait_scatter(b - _LOOK, b - _LOOK)
        start_gather(b + _LOOK, (b + _LOOK) % _NBUF)

    def round_body(r, _):
        for b in range(_NBUF):
            j = r * _NBUF + b
            wait_gather(j, b)
            start_scatter(j, b)
            wait_scatter(j - _LOOK, (b - _LOOK) % _NBUF)
            start_gather(j + _LOOK, (b + _LOOK) % _NBUF)
        return 0

    lax.fori_loop(1, _NCHUNK // _NBUF - 1, round_body, 0)

    r = _NCHUNK // _NBUF - 1
    for b in range(_NBUF):
        j = r * _NBUF + b
        wait_gather(j, b)
        start_scatter(j, b)
        wait_scatter(j - _LOOK, (b - _LOOK) % _NBUF)
        if j + _LOOK < _NCHUNK:
            start_gather(j + _LOOK, (b + _LOOK) % _NBUF)

    for b in range(_NBUF - _LOOK, _NBUF):
        j = r * _NBUF + b
        wait_scatter(j, b)


def _retile_body(i_ref, o_ref):
    x = i_ref[...]
    x3 = x.reshape(512, 2, 64)
    y = jnp.transpose(x3, (2, 0, 1))
    o_ref[...] = y.reshape(1, DIM, 1024)


def _retile(res2):
    nq = BATCH // 1024
    return pl.pallas_call(
        _retile_body,
        grid=(HIST, nq),
        in_specs=[pl.BlockSpec((512, 128), lambda h, q: (h * nq + q, 0))],
        out_specs=pl.BlockSpec((1, DIM, 1024), lambda h, q: (h, 0, q)),
        out_shape=jax.ShapeDtypeStruct((HIST, DIM, BATCH), jnp.float32),
    )(res2)


def kernel(words, weight):
    keep = jax.random.bernoulli(
        jax.random.key(42), 1.0 - EMBED_P, (VOCAB,)
    ).astype(weight.dtype)
    mask_lanes = (keep / (1.0 - EMBED_P)).reshape(VOCAB // _MROWS, 1, _MROWS)
    masked = _masked_table(weight, mask_lanes)
    idx = words.astype(jnp.int32).T.reshape(_NW, _NCHUNK, _CH)
    out = _sc_gather(masked, idx)
    res2 = out.reshape(_B * DIM // 128, 128)
    planes = _retile(res2)
    return jnp.transpose(planes, (2, 0, 1))


# speedup vs baseline: 9.3125x; 9.3125x over previous
import functools

import jax
import jax.numpy as jnp
from jax import lax
from jax.experimental import pallas as pl
from jax.experimental.pallas import tpu as pltpu
from jax.experimental.pallas import tpu_sc as plsc

VOCAB = 100000
DIM = 64
EMBED_P = 0.1
BATCH = 4096
HIST = 200

_B = BATCH * HIST

_info = plsc.get_sparse_core_info()
_NC = _info.num_cores
_NS = _info.num_subcores
_NW = _NC * _NS
_BPW = _B // _NW
_CH = 128
_NCHUNK = _BPW // _CH
_NBUF = 8
_LOOK = 4

_MROWS = 4000


def _scale_body(w_ref, m_ref, o_ref):
    m_row = m_ref[...].reshape(1, _MROWS)
    m_col = lax.transpose(m_row, (1, 0))
    o_ref[...] = w_ref[...] * m_col


def _masked_table(weight, mask_lanes):
    grid = VOCAB // _MROWS
    return pl.pallas_call(
        _scale_body,
        grid=(grid,),
        in_specs=[
            pl.BlockSpec((_MROWS, DIM), lambda i: (i, 0)),
            pl.BlockSpec((1, 1, _MROWS), lambda i: (i, 0, 0)),
        ],
        out_specs=pl.BlockSpec((_MROWS, DIM), lambda i: (i, 0)),
        out_shape=jax.ShapeDtypeStruct((VOCAB, DIM), jnp.float32),
    )(weight, mask_lanes)


_mesh = plsc.VectorSubcoreMesh(core_axis_name="c", subcore_axis_name="s")


@functools.partial(
    pl.kernel,
    mesh=_mesh,
    out_type=jax.ShapeDtypeStruct((_B * DIM // 128, 128), jnp.float32),
    scratch_types=[
        pltpu.VMEM((_NCHUNK, _CH), jnp.int32),
    ]
    + [pltpu.VMEM((_CH, DIM), jnp.float32) for _ in range(_NBUF)]
    + [pltpu.SemaphoreType.DMA for _ in range(2 * _NBUF)],
    compiler_params=pltpu.CompilerParams(use_tc_tiling_on_sc=False),
)
def _sc_gather(tab_hbm, idx_hbm, out_hbm, idx_v, *bufs_and_sems):
    rows = bufs_and_sems[:_NBUF]
    gsem = bufs_and_sems[_NBUF:2 * _NBUF]
    ssem = bufs_and_sems[2 * _NBUF:]
    wid = lax.axis_index("s") * _NC + lax.axis_index("c")
    base_c = wid * _NCHUNK
    pltpu.sync_copy(idx_hbm.at[wid], idx_v)

    def start_gather(j, b):
        pltpu.async_copy(tab_hbm.at[idx_v.at[j]], rows[b], gsem[b])

    def wait_gather(j, b):
        pltpu.make_async_copy(tab_hbm.at[idx_v.at[j]], rows[b], gsem[b]).wait()

    def _dst(j):
        c = base_c + j
        h = c // (BATCH // _CH)
        r32 = c % (BATCH // _CH)
        row0 = h * (BATCH // 2) + (r32 // 8) * 512 + (r32 % 4) * _CH
        par = (r32 // 4) % 2
        return row0, par

    def start_scatter(j, b):
        row0, par = _dst(j)
        pltpu.async_copy(
            rows[b],
            out_hbm.at[pl.ds(row0, _CH), pl.ds(par * DIM, DIM)],
            ssem[b],
        )

    def wait_scatter(j, b):
        row0, par = _dst(j)
        pltpu.make_async_copy(
            rows[b],
            out_hbm.at[pl.ds(row0, _CH), pl.ds(par * DIM, DIM)],
            ssem[b],
        ).wait()

    for b in range(_LOOK):
        start_gather(b, b)

    for b in range(_NBUF):
        wait_gather(b, b)
        start_scatter(b, b)
        if b >= _LOOK:
            wait_scatter(b - _LOOK, b - _LOOK)
        start_gather(b + _LOOK, (b + _LOOK) % _NBUF)

    def round_body(r, _):
        for b in range(_NBUF):
            j = r * _NBUF + b
            wait_gather(j, b)
            start_scatter(j, b)
            wait_scatter(j - _LOOK, (b - _LOOK) % _NBUF)
            start_gather(j + _LOOK, (b + _LOOK) % _NBUF)
        return 0

    lax.fori_loop(1, _NCHUNK // _NBUF - 1, round_body, 0)

    r = _NCHUNK // _NBUF - 1
    for b in range(_NBUF):
        j = r * _NBUF + b
        wait_gather(j, b)
        start_scatter(j, b)
        wait_scatter(j - _LOOK, (b - _LOOK) % _NBUF)
        if j + _LOOK < _NCHUNK:
            start_gather(j + _LOOK, (b + _LOOK) % _NBUF)

    for b in range(_NBUF - _LOOK, _NBUF):
        j = r * _NBUF + b
        wait_scatter(j, b)


def _retile_body(i_ref, o_ref):
    x = i_ref[...]
    t = lax.transpose(x, (1, 0))
    o_ref[...] = jnp.concatenate([t[0:DIM, :], t[DIM:, :]], axis=1)[None]


def _retile(res2):
    nq = BATCH // 1024
    return pl.pallas_call(
        _retile_body,
        grid=(HIST, nq),
        in_specs=[pl.BlockSpec((512, 128), lambda h, q: (h * nq + q, 0))],
        out_specs=pl.BlockSpec((1, DIM, 1024), lambda h, q: (h, 0, q)),
        out_shape=jax.ShapeDtypeStruct((HIST, DIM, BATCH), jnp.float32),
    )(res2)


def kernel(words, weight):
    keep = jax.random.bernoulli(
        jax.random.key(42), 1.0 - EMBED_P, (VOCAB,)
    ).astype(weight.dtype)
    mask_lanes = (keep / (1.0 - EMBED_P)).reshape(VOCAB // _MROWS, 1, _MROWS)
    masked = _masked_table(weight, mask_lanes)
    idx = words.astype(jnp.int32).T.reshape(_NW, _NCHUNK, _CH)
    out = _sc_gather(masked, idx)
    res2 = out.reshape(_B * DIM // 128, 128)
    planes = _retile(res2)
    return jnp.transpose(planes, (2, 0, 1))
